# Initial kernel scaffold; baseline (speedup 1.0000x reference)
import jax
import jax.numpy as jnp
from jax.experimental import pallas as pl


def kernel(input_ids, token_type_ids, word_table, pos_table, type_table, gamma, beta):
    raise NotImplementedError("write your pallas kernel here")


# speedup vs baseline: 3.7425x; 3.7425x over previous
import functools

import jax
import jax.numpy as jnp
from jax import lax
from jax.experimental import pallas as pl
from jax.experimental.pallas import tpu as pltpu
from jax.experimental.pallas import tpu_sc as plsc


def _sc_gather(table, ids2d):
    info = plsc.get_sparse_core_info()
    nw = info.num_cores * info.num_subcores
    n_rows, idx_w = ids2d.shape
    rows_per_w = n_rows // nw
    d = table.shape[1]
    mesh = plsc.VectorSubcoreMesh(core_axis_name="c", subcore_axis_name="s")

    @functools.partial(
        pl.kernel,
        mesh=mesh,
        out_type=jax.ShapeDtypeStruct((n_rows * idx_w, d), jnp.float32),
        scratch_types=[
            pltpu.VMEM((rows_per_w, idx_w), jnp.int32),
            pltpu.VMEM((idx_w, d), jnp.float32),
            pltpu.VMEM((idx_w, d), jnp.float32),
            pltpu.SemaphoreType.DMA,
            pltpu.SemaphoreType.DMA,
        ],
    )
    def k(table_hbm, ids_hbm, out_hbm, idx_v, buf0, buf1, sem0, sem1):
        wid = lax.axis_index("s") * info.num_cores + lax.axis_index("c")
        row0 = wid * rows_per_w
        pltpu.sync_copy(ids_hbm.at[pl.ds(row0, rows_per_w)], idx_v)
        bufs = (buf0, buf1)
        sems = (sem0, sem1)
        cps = [None, None]
        cps[0] = pltpu.async_copy(table_hbm.at[idx_v.at[0]], buf0, sem0)
        for j in range(rows_per_w):
            nj = j + 1
            if nj < rows_per_w:
                cps[nj % 2] = pltpu.async_copy(
                    table_hbm.at[idx_v.at[nj]], bufs[nj % 2], sems[nj % 2]
                )
            cps[j % 2].wait()
            pltpu.sync_copy(bufs[j % 2], out_hbm.at[pl.ds((row0 + j) * idx_w, idx_w)])

    return k(table, ids2d)


def _tc_ln(words, pos_table, tt_f32, type_table, gamma, beta):
    t, d = words.shape
    s = pos_table.shape[0]
    bs = 1024
    nblk = t // bs
    pos_rep = s // bs

    def body(w_ref, p_ref, tt_ref, ty_ref, g_ref, b_ref, o_ref):
        w = w_ref[...]
        p = p_ref[...]
        tt = tt_ref[...]
        ty = ty_ref[...]
        type_row = jnp.where(tt > 0.5, ty[1][None, :], ty[0][None, :])
        x = w + p + type_row
        mean = jnp.mean(x, axis=-1, keepdims=True)
        xc = x - mean
        var = jnp.mean(xc * xc, axis=-1, keepdims=True)
        y = xc * lax.rsqrt(var + 1e-12)
        o_ref[...] = y * g_ref[...] + b_ref[...]

    return pl.pallas_call(
        body,
        grid=(nblk,),
        in_specs=[
            pl.BlockSpec((bs, d), lambda i: (i, 0)),
            pl.BlockSpec((bs, d), lambda i: (i % pos_rep, 0)),
            pl.BlockSpec((bs, 1), lambda i: (i, 0)),
            pl.BlockSpec((2, d), lambda i: (0, 0)),
            pl.BlockSpec((1, d), lambda i: (0, 0)),
            pl.BlockSpec((1, d), lambda i: (0, 0)),
        ],
        out_specs=pl.BlockSpec((bs, d), lambda i: (i, 0)),
        out_shape=jax.ShapeDtypeStruct((t, d), jnp.float32),
    )(words, pos_table, tt_f32, type_table, gamma, beta)


def kernel(input_ids, token_type_ids, word_table, pos_table, type_table, gamma, beta):
    b, s = input_ids.shape
    t = b * s
    d = word_table.shape[1]
    ids2d = input_ids.reshape(t // 128, 128).astype(jnp.int32)
    words = _sc_gather(word_table, ids2d)
    tt = token_type_ids.reshape(t, 1).astype(jnp.float32)
    out = _tc_ln(
        words,
        pos_table,
        tt,
        type_table,
        gamma.reshape(1, d),
        beta.reshape(1, d),
    )
    return out.reshape(b, s, d)
